# baseline (speedup 1.0000x reference)
import jax
import jax.numpy as jnp
from jax.experimental import pallas as pl
from jax.experimental.pallas import tpu as pltpu

NEG = -3e38
ROWS = 8


def _shift_right(x, d, fill):
    rolled = jnp.roll(x, d, axis=-1)
    lane = jax.lax.broadcasted_iota(jnp.int32, x.shape, x.ndim - 1)
    return jnp.where(lane < d, fill, rolled)


def _cummax_j(x):
    for d in (1, 2, 4, 8, 16, 32, 64):
        x = jnp.maximum(x, _shift_right(x, d, NEG))
    return x


def _row_update(row, half_a, b, M):
    half_b = 0.5 * b
    S = jax.lax.dot_general(
        half_b, M, (((1,), (0,)), ((), ())),
        preferred_element_type=jnp.float32,
    )
    tmp = row + half_a
    base = jnp.maximum(tmp, _shift_right(tmp, 1, NEG)) + half_b
    return S + _cummax_j(base - S), half_b


def _dp_kernel(nb, nt, img_ref, m_ref, out_ref, row_ref, prev_ref, buf_ref, sem):
    B, I, J = img_ref.shape
    Bb = row_ref.shape[0]
    b = pl.program_id(0)
    t = pl.program_id(1)
    M = m_ref[:, :]

    def strip_copies(bi, ti, slot):
        return [
            pltpu.make_async_copy(
                img_ref.at[pl.ds(bi * Bb, Bb), ti * ROWS + r, :],
                buf_ref.at[slot, :, pl.ds(r * J, J)],
                sem.at[slot, r],
            )
            for r in range(ROWS)
        ]

    @pl.when(jnp.logical_and(b == 0, t == 0))
    def _start_first():
        for c in strip_copies(0, 0, 0):
            c.start()

    nxt = t + 1
    nb_i = jnp.where(nxt == nt, b + 1, b)
    nt_i = jnp.where(nxt == nt, 0, nxt)

    @pl.when(nb_i < nb)
    def _prefetch():
        for c in strip_copies(nb_i, nt_i, nxt % 2):
            c.start()

    slot = t % 2
    for c in strip_copies(b, t, slot):
        c.wait()

    def img_row(r):
        return buf_ref[slot, :, r * J : (r + 1) * J]

    @pl.when(t == 0)
    def _init():
        r0 = img_row(0)
        half_r0 = 0.5 * r0
        S0 = jax.lax.dot_general(
            half_r0, M, (((1,), (0,)), ((), ())),
            preferred_element_type=jnp.float32,
        )
        row = S0 + (r0[:, 0:1] - S0[:, 0:1])
        half_a = half_r0
        for r in range(1, ROWS):
            row, half_a = _row_update(row, half_a, img_row(r), M)
        row_ref[:, :] = row
        prev_ref[:, :] = half_a

    @pl.when(t != 0)
    def _step():
        row = row_ref[:, :]
        half_a = prev_ref[:, :]
        for r in range(ROWS):
            row, half_a = _row_update(row, half_a, img_row(r), M)
        row_ref[:, :] = row
        prev_ref[:, :] = half_a

    out_ref[:, :] = row_ref[:, J - 1 : J]


@jax.jit
def kernel(images):
    import functools

    B, I, J = images.shape
    Bb = 256
    nb = B // Bb
    nt = I // ROWS
    k = jnp.arange(J)
    M = (2.0 * (k[:, None] < k[None, :]) + (k[:, None] == k[None, :])).astype(
        jnp.float32
    )
    out = pl.pallas_call(
        functools.partial(_dp_kernel, nb, nt),
        grid=(nb, nt),
        in_specs=[
            pl.BlockSpec(memory_space=pl.ANY),
            pl.BlockSpec((J, J), lambda b, t: (0, 0)),
        ],
        out_specs=pl.BlockSpec((Bb, 1), lambda b, t: (b, 0)),
        out_shape=jax.ShapeDtypeStruct((B, 1), jnp.float32),
        scratch_shapes=[
            pltpu.VMEM((Bb, J), jnp.float32),
            pltpu.VMEM((Bb, J), jnp.float32),
            pltpu.VMEM((2, Bb, ROWS * J), jnp.float32),
            pltpu.SemaphoreType.DMA((2, ROWS)),
        ],
        compiler_params=pltpu.CompilerParams(
            dimension_semantics=("arbitrary", "arbitrary"),
        ),
    )(images, M)
    return out[:, 0]


# speedup vs baseline: 2.0210x; 2.0210x over previous
import jax
import jax.numpy as jnp
from jax.experimental import pallas as pl
from jax.experimental.pallas import tpu as pltpu

NEG = -3e38
ROWS = 8


def _shift_right(x, d, fill):
    rolled = jnp.roll(x, d, axis=-1)
    lane = jax.lax.broadcasted_iota(jnp.int32, x.shape, x.ndim - 1)
    return jnp.where(lane < d, fill, rolled)


def _cummax_j(x):
    for d in (1, 2, 4, 8, 16, 32, 64):
        x = jnp.maximum(x, _shift_right(x, d, NEG))
    return x


def _row_update(row, half_a, b, M):
    half_b = 0.5 * b
    S = jax.lax.dot_general(
        half_b, M, (((1,), (0,)), ((), ())),
        preferred_element_type=jnp.float32,
    )
    tmp = row + half_a
    base = jnp.maximum(tmp, _shift_right(tmp, 1, NEG)) + half_b
    return S + _cummax_j(base - S), half_b


def _dp_kernel(nb, nt, img_ref, m_ref, out_ref, row_ref, prev_ref, buf_ref, sem):
    B, I, J = img_ref.shape
    Bb = row_ref.shape[0]
    b = pl.program_id(0)
    t = pl.program_id(1)
    M = m_ref[:, :]

    def strip_copies(bi, ti, slot):
        return [
            pltpu.make_async_copy(
                img_ref.at[pl.ds(bi * Bb, Bb), ti * ROWS + r, :],
                buf_ref.at[slot, :, pl.ds(r * J, J)],
                sem.at[slot, r],
            )
            for r in range(ROWS)
        ]

    @pl.when(jnp.logical_and(b == 0, t == 0))
    def _start_first():
        for c in strip_copies(0, 0, 0):
            c.start()

    nxt = t + 1
    nb_i = jnp.where(nxt == nt, b + 1, b)
    nt_i = jnp.where(nxt == nt, 0, nxt)

    @pl.when(nb_i < nb)
    def _prefetch():
        for c in strip_copies(nb_i, nt_i, nxt % 2):
            c.start()

    slot = t % 2
    for c in strip_copies(b, t, slot):
        c.wait()

    def img_row(r):
        return buf_ref[slot, :, r * J : (r + 1) * J]

    @pl.when(t == 0)
    def _init():
        r0 = img_row(0)
        half_r0 = 0.5 * r0
        S0 = jax.lax.dot_general(
            half_r0, M, (((1,), (0,)), ((), ())),
            preferred_element_type=jnp.float32,
        )
        row = S0 + (r0[:, 0:1] - S0[:, 0:1])
        half_a = half_r0
        for r in range(1, ROWS):
            row, half_a = _row_update(row, half_a, img_row(r), M)
        row_ref[:, :] = row
        prev_ref[:, :] = half_a

    @pl.when(t != 0)
    def _step():
        row = row_ref[:, :]
        half_a = prev_ref[:, :]
        for r in range(ROWS):
            row, half_a = _row_update(row, half_a, img_row(r), M)
        row_ref[:, :] = row
        prev_ref[:, :] = half_a

    out_ref[:, :] = row_ref[:, J - 1 : J]


@jax.jit
def kernel(images):
    import functools

    B, I, J = images.shape
    Bb = 1024
    nb = B // Bb
    nt = I // ROWS
    k = jnp.arange(J)
    M = (2.0 * (k[:, None] < k[None, :]) + (k[:, None] == k[None, :])).astype(
        jnp.float32
    )
    out = pl.pallas_call(
        functools.partial(_dp_kernel, nb, nt),
        grid=(nb, nt),
        in_specs=[
            pl.BlockSpec(memory_space=pl.ANY),
            pl.BlockSpec((J, J), lambda b, t: (0, 0)),
        ],
        out_specs=pl.BlockSpec((Bb, 1), lambda b, t: (b, 0)),
        out_shape=jax.ShapeDtypeStruct((B, 1), jnp.float32),
        scratch_shapes=[
            pltpu.VMEM((Bb, J), jnp.float32),
            pltpu.VMEM((Bb, J), jnp.float32),
            pltpu.VMEM((2, Bb, ROWS * J), jnp.float32),
            pltpu.SemaphoreType.DMA((2, ROWS)),
        ],
        compiler_params=pltpu.CompilerParams(
            dimension_semantics=("arbitrary", "arbitrary"),
        ),
    )(images, M)
    return out[:, 0]


# speedup vs baseline: 2.0426x; 1.0107x over previous
import jax
import jax.numpy as jnp
from jax.experimental import pallas as pl
from jax.experimental.pallas import tpu as pltpu

NEG = -3e38
ROWS = 16


def _shift_right(x, d, fill):
    rolled = jnp.roll(x, d, axis=-1)
    lane = jax.lax.broadcasted_iota(jnp.int32, x.shape, x.ndim - 1)
    return jnp.where(lane < d, fill, rolled)


def _cummax_j(x):
    for d in (1, 2, 4, 8, 16, 32, 64):
        x = jnp.maximum(x, _shift_right(x, d, NEG))
    return x


def _row_update(row, half_a, b, M):
    half_b = 0.5 * b
    S = jax.lax.dot_general(
        half_b, M, (((1,), (0,)), ((), ())),
        preferred_element_type=jnp.float32,
    )
    tmp = row + half_a
    base = jnp.maximum(tmp, _shift_right(tmp, 1, NEG)) + half_b
    return S + _cummax_j(base - S), half_b


def _dp_kernel(nb, nt, img_ref, m_ref, out_ref, row_ref, prev_ref, buf_ref, sem):
    B, I, J = img_ref.shape
    Bb = row_ref.shape[0]
    b = pl.program_id(0)
    t = pl.program_id(1)
    M = m_ref[:, :]

    def strip_copies(bi, ti, slot):
        return [
            pltpu.make_async_copy(
                img_ref.at[pl.ds(bi * Bb, Bb), ti * ROWS + r, :],
                buf_ref.at[slot, :, pl.ds(r * J, J)],
                sem.at[slot, r],
            )
            for r in range(ROWS)
        ]

    @pl.when(jnp.logical_and(b == 0, t == 0))
    def _start_first():
        for c in strip_copies(0, 0, 0):
            c.start()

    nxt = t + 1
    nb_i = jnp.where(nxt == nt, b + 1, b)
    nt_i = jnp.where(nxt == nt, 0, nxt)

    @pl.when(nb_i < nb)
    def _prefetch():
        for c in strip_copies(nb_i, nt_i, nxt % 2):
            c.start()

    slot = t % 2
    for c in strip_copies(b, t, slot):
        c.wait()

    def img_row(r):
        return buf_ref[slot, :, r * J : (r + 1) * J]

    @pl.when(t == 0)
    def _init():
        r0 = img_row(0)
        half_r0 = 0.5 * r0
        S0 = jax.lax.dot_general(
            half_r0, M, (((1,), (0,)), ((), ())),
            preferred_element_type=jnp.float32,
        )
        row = S0 + (r0[:, 0:1] - S0[:, 0:1])
        half_a = half_r0
        for r in range(1, ROWS):
            row, half_a = _row_update(row, half_a, img_row(r), M)
        row_ref[:, :] = row
        prev_ref[:, :] = half_a

    @pl.when(t != 0)
    def _step():
        row = row_ref[:, :]
        half_a = prev_ref[:, :]
        for r in range(ROWS):
            row, half_a = _row_update(row, half_a, img_row(r), M)
        row_ref[:, :] = row
        prev_ref[:, :] = half_a

    out_ref[:, :] = row_ref[:, J - 1 : J]


@jax.jit
def kernel(images):
    import functools

    B, I, J = images.shape
    Bb = 1024
    nb = B // Bb
    nt = I // ROWS
    k = jnp.arange(J)
    M = (2.0 * (k[:, None] < k[None, :]) + (k[:, None] == k[None, :])).astype(
        jnp.float32
    )
    out = pl.pallas_call(
        functools.partial(_dp_kernel, nb, nt),
        grid=(nb, nt),
        in_specs=[
            pl.BlockSpec(memory_space=pl.ANY),
            pl.BlockSpec((J, J), lambda b, t: (0, 0)),
        ],
        out_specs=pl.BlockSpec((Bb, 1), lambda b, t: (b, 0)),
        out_shape=jax.ShapeDtypeStruct((B, 1), jnp.float32),
        scratch_shapes=[
            pltpu.VMEM((Bb, J), jnp.float32),
            pltpu.VMEM((Bb, J), jnp.float32),
            pltpu.VMEM((2, Bb, ROWS * J), jnp.float32),
            pltpu.SemaphoreType.DMA((2, ROWS)),
        ],
        compiler_params=pltpu.CompilerParams(
            dimension_semantics=("arbitrary", "arbitrary"),
        ),
    )(images, M)
    return out[:, 0]
